# baseline (speedup 1.0000x reference)
import functools

import jax
import jax.numpy as jnp
from jax import lax
from jax.experimental import pallas as pl
from jax.experimental.pallas import tpu as pltpu
from jax.experimental.pallas import tpu_sc as plsc

VOCAB = 1000000
EMBED_DIM = 64
BATCH = 4096
SEQ = 200

NC = 2
NS = 16
NW = NC * NS
RPW = BATCH // NW
SEQ_HI = SEQ // 8
NKV = EMBED_DIM // 16


def _encoder_kernel(ids_hbm, table_hbm, out_hbm,
                    idx_v, buf0, buf1, out_v, sem0, sem1):
    wid = lax.axis_index("s") * NC + lax.axis_index("c")

    inv = jnp.float32(1.0 / SEQ)
    bufs = (buf0, buf1)
    sems = (sem0, sem1)

    pltpu.sync_copy(ids_hbm.at[:, wid], idx_v)

    def zero_body(i, carry):
        z = jnp.zeros((16,), jnp.float32)
        for k in range(NKV):
            out_v[i, pl.ds(16 * k, 16)] = z
        return carry

    lax.fori_loop(0, RPW, zero_body, 0)

    def fire(hi, lo, slot):
        pltpu.async_copy(
            table_hbm.at[idx_v.at[hi, lo]], bufs[slot], sems[slot])

    def wait(slot):
        pltpu.make_async_copy(
            table_hbm.at[idx_v.at[0, 0]], bufs[slot], sems[slot]).wait()

    def accum(slot):
        buf = bufs[slot]

        def acc_body(i, carry):
            for k in range(NKV):
                plsc.addupdate(
                    out_v.at[i, pl.ds(16 * k, 16)], buf[i, pl.ds(16 * k, 16)])
            return carry

        lax.fori_loop(0, RPW, acc_body, 0, unroll=4)

    fire(0, 0, 0)

    def outer(hi, carry):
        for lo in range(8):
            slot = lo % 2
            lo_n = (lo + 1) % 8
            hi_n = hi + (lo + 1) // 8

            @pl.when(hi_n < SEQ_HI)
            def _():
                fire(hi_n, lo_n, 1 - slot)

            wait(slot)
            accum(slot)
        return carry

    lax.fori_loop(0, SEQ_HI, outer, 0)

    def scale_body(i, carry):
        for k in range(NKV):
            out_v[i, pl.ds(16 * k, 16)] = out_v[i, pl.ds(16 * k, 16)] * inv
        return carry

    lax.fori_loop(0, RPW, scale_body, 0)
    pltpu.sync_copy(out_v, out_hbm.at[pl.ds(wid * RPW, RPW)])


def kernel(text_ids, table):
    ids = text_ids.astype(jnp.int32)
    ids4d = ids.T.reshape(SEQ_HI, 8, NW, RPW).transpose(0, 2, 1, 3)
    mesh = plsc.VectorSubcoreMesh(core_axis_name="c", subcore_axis_name="s")
    k = functools.partial(
        pl.kernel,
        mesh=mesh,
        out_type=jax.ShapeDtypeStruct((BATCH, EMBED_DIM), jnp.float32),
        scratch_types=[
            pltpu.VMEM((SEQ_HI, 8, RPW), jnp.int32),
            pltpu.VMEM((RPW, EMBED_DIM), jnp.float32),
            pltpu.VMEM((RPW, EMBED_DIM), jnp.float32),
            pltpu.VMEM((RPW, EMBED_DIM), jnp.float32),
            pltpu.SemaphoreType.DMA,
            pltpu.SemaphoreType.DMA,
        ],
        compiler_params=pltpu.CompilerParams(use_tc_tiling_on_sc=False),
    )(_encoder_kernel)
    return k(ids4d, table)
